# baseline (speedup 1.0000x reference)
import math

import jax
import jax.numpy as jnp
from jax import lax
from jax.experimental import pallas as pl
from jax.experimental.pallas import tpu as pltpu

_EPS = 1e-12
_F32 = jnp.float32
_BF16 = jnp.bfloat16


def _dot_t(a, b):
    return lax.dot_general(a, b, (((1,), (1,)), ((), ())),
                           preferred_element_type=_F32)


def _gelu(v):
    inner = math.sqrt(2.0 / math.pi) * (v + 0.044715 * v * v * v)
    return 0.5 * v * (1.0 + jnp.tanh(inner))


def kernel(x, W1w, W1b, W2w, W2b, Wfc, bfc, Wproj, bproj, Wr, br, g1, be1, g2, be2):
    Bb, N, D = x.shape
    E, F, _ = Wfc.shape
    H = 12
    DH = D // H
    K = 2
    TILE = 256
    NT = N // TILE
    ND = float(N * D)
    scale = 1.0 / math.sqrt(float(N))

    x2 = x.reshape(N, D)
    xb = x2.astype(_BF16)
    W1b16 = W1w.astype(_BF16)
    W2b16 = W2w.astype(_BF16)
    b1r = W1b.reshape(1, 3 * D)
    b2r = W2b.reshape(1, D)
    g1r, be1r = g1.reshape(1, D), be1.reshape(1, D)
    g2r, be2r = g2.reshape(1, D), be2.reshape(1, D)
    brr = br.reshape(1, E)

    def _qkv(x_ref, w1_ref, b1_ref, v_ref, s_ref):
        i = pl.program_id(0)
        xp = _dot_t(x_ref[...], w1_ref[...]) + b1_ref[...]
        v_ref[...] = xp[:, 2 * D:].astype(_BF16)

        @pl.when(i == 0)
        def _():
            s_ref[...] = jnp.zeros_like(s_ref)

        xpb = xp.astype(_BF16)
        for h in range(H):
            qh = xpb[:, h * DH:(h + 1) * DH]
            kh = xpb[:, D + h * DH:D + (h + 1) * DH]
            s_ref[h] += lax.dot_general(qh, kh, (((0,), (0,)), ((), ())),
                                        preferred_element_type=_F32)

    V, S = pl.pallas_call(
        _qkv,
        grid=(NT,),
        in_specs=[
            pl.BlockSpec((TILE, D), lambda i: (i, 0)),
            pl.BlockSpec((3 * D, D), lambda i: (0, 0)),
            pl.BlockSpec((1, 3 * D), lambda i: (0, 0)),
        ],
        out_specs=[
            pl.BlockSpec((TILE, D), lambda i: (i, 0)),
            pl.BlockSpec((H, DH, DH), lambda i: (0, 0, 0)),
        ],
        out_shape=[
            jax.ShapeDtypeStruct((N, D), _BF16),
            jax.ShapeDtypeStruct((H, DH, DH), _F32),
        ],
    )(xb, W1b16, b1r)

    def _attn(s_ref, v_ref, o_ref):
        w = jax.nn.softmax(s_ref[...] * scale, axis=-1).astype(_BF16)
        vt = v_ref[...]
        for h in range(H):
            oh = lax.dot_general(w[h], vt[:, h * DH:(h + 1) * DH],
                                 (((1,), (1,)), ((), ())),
                                 preferred_element_type=_F32)
            o_ref[h * DH:(h + 1) * DH, :] = oh.astype(_BF16)

    out_hdn = pl.pallas_call(
        _attn,
        grid=(NT,),
        in_specs=[
            pl.BlockSpec((H, DH, DH), lambda i: (0, 0, 0)),
            pl.BlockSpec((TILE, D), lambda i: (i, 0)),
        ],
        out_specs=pl.BlockSpec((D, TILE), lambda i: (0, i)),
        out_shape=jax.ShapeDtypeStruct((D, N), _BF16),
    )(S, V)

    att = out_hdn.reshape(N, D)

    def _proj(a_ref, w2_ref, b2_ref, x_ref, g1_ref, be1_ref, wr_ref, br_ref,
              y_ref, cs_ref, css_ref, stats_ref, idx_ref):
        i = pl.program_id(0)
        y = _dot_t(a_ref[...], w2_ref[...]) + b2_ref[...] + x_ref[...]
        y_ref[...] = y

        @pl.when(i == 0)
        def _():
            cs_ref[...] = jnp.zeros_like(cs_ref)
            css_ref[...] = jnp.zeros_like(css_ref)

        cs_ref[...] += jnp.sum(y, axis=0, keepdims=True)
        css_ref[...] += jnp.sum(y * y, axis=0, keepdims=True)

        @pl.when(i == NT - 1)
        def _():
            total = jnp.sum(cs_ref[...])
            mu = total / ND
            ssq = jnp.sum(css_ref[...])
            var = (ssq - ND * mu * mu) / (ND - 1.0)
            rstd = 1.0 / jnp.sqrt(var + _EPS)
            colmean_x1 = ((cs_ref[...] / N) - mu) * rstd * g1_ref[...] \
                + be1_ref[...]
            logits = _dot_t(colmean_x1, wr_ref[...]) + br_ref[...]
            probs = jax.nn.softmax(logits, axis=-1)
            iota_e = lax.broadcasted_iota(jnp.int32, (1, E), 1)
            v0 = jnp.max(probs, axis=1, keepdims=True)
            i0 = jnp.argmax(probs, axis=1).astype(jnp.int32)[:, None]
            masked = jnp.where(iota_e == i0, -jnp.inf, probs)
            v1 = jnp.max(masked, axis=1, keepdims=True)
            i1 = jnp.argmax(masked, axis=1).astype(jnp.int32)[:, None]
            iota4 = lax.broadcasted_iota(jnp.int32, (1, 4), 1)
            stats_ref[...] = jnp.where(
                iota4 == 0, mu,
                jnp.where(iota4 == 1, rstd, jnp.where(iota4 == 2, v0, v1)))
            iota2 = lax.broadcasted_iota(jnp.int32, (1, K), 1)
            idx_ref[...] = jnp.where(iota2 == 0, i0, i1)

    y, cs, css, stats, idx2 = pl.pallas_call(
        _proj,
        grid=(NT,),
        in_specs=[
            pl.BlockSpec((TILE, D), lambda i: (i, 0)),
            pl.BlockSpec((D, D), lambda i: (0, 0)),
            pl.BlockSpec((1, D), lambda i: (0, 0)),
            pl.BlockSpec((TILE, D), lambda i: (i, 0)),
            pl.BlockSpec((1, D), lambda i: (0, 0)),
            pl.BlockSpec((1, D), lambda i: (0, 0)),
            pl.BlockSpec((E, D), lambda i: (0, 0)),
            pl.BlockSpec((1, E), lambda i: (0, 0)),
        ],
        out_specs=[
            pl.BlockSpec((TILE, D), lambda i: (i, 0)),
            pl.BlockSpec((1, D), lambda i: (0, 0)),
            pl.BlockSpec((1, D), lambda i: (0, 0)),
            pl.BlockSpec((1, 4), lambda i: (0, 0)),
            pl.BlockSpec((1, K), lambda i: (0, 0)),
        ],
        out_shape=[
            jax.ShapeDtypeStruct((N, D), _F32),
            jax.ShapeDtypeStruct((1, D), _F32),
            jax.ShapeDtypeStruct((1, D), _F32),
            jax.ShapeDtypeStruct((1, 4), _F32),
            jax.ShapeDtypeStruct((1, K), jnp.int32),
        ],
    )(att, W2b16, b2r, x2, g1r, be1r, Wr, brr)

    idx_flat = idx2.reshape(K)
    y01 = y[:K].reshape(K, 1, D)
    bfc3 = bfc.reshape(E, 1, F)
    bproj3 = bproj.reshape(E, 1, D)

    FT = 1536
    NF = F // FT

    def _expert(idx_ref, y01_ref, stats_ref, g1_ref, be1_ref,
                wfc_ref, bfc_ref, wproj_ref, bproj_ref, m_ref):
        j = pl.program_id(0)
        f = pl.program_id(1)
        mu = stats_ref[:, 0:1]
        rstd = stats_ref[:, 1:2]
        val = jnp.where(j == 0, stats_ref[:, 2:3], stats_ref[:, 3:4])
        x1j = (y01_ref[0] - mu) * rstd * g1_ref[...] + be1_ref[...]
        h = _gelu(_dot_t(x1j, wfc_ref[0]) + bfc_ref[0])
        o = _dot_t(h, wproj_ref[0])

        @pl.when((j == 0) & (f == 0))
        def _():
            m_ref[...] = jnp.zeros_like(m_ref)

        m_ref[...] += val * o

        @pl.when(f == 0)
        def _():
            m_ref[...] += val * bproj_ref[0]

    m = pl.pallas_call(
        _expert,
        grid_spec=pltpu.PrefetchScalarGridSpec(
            num_scalar_prefetch=1,
            grid=(K, NF),
            in_specs=[
                pl.BlockSpec((1, 1, D), lambda j, f, idx: (j, 0, 0)),
                pl.BlockSpec((1, 4), lambda j, f, idx: (0, 0)),
                pl.BlockSpec((1, D), lambda j, f, idx: (0, 0)),
                pl.BlockSpec((1, D), lambda j, f, idx: (0, 0)),
                pl.BlockSpec((1, FT, D), lambda j, f, idx: (idx[j], f, 0)),
                pl.BlockSpec((1, 1, FT), lambda j, f, idx: (idx[j], 0, f)),
                pl.BlockSpec((1, D, FT), lambda j, f, idx: (idx[j], 0, f)),
                pl.BlockSpec((1, 1, D), lambda j, f, idx: (idx[j], 0, 0)),
            ],
            out_specs=pl.BlockSpec((1, D), lambda j, f, idx: (0, 0)),
        ),
        out_shape=jax.ShapeDtypeStruct((1, D), _F32),
    )(idx_flat, y01, stats, g1r, be1r, Wfc, bfc3, Wproj, bproj3)

    def _final(y_ref, cs_ref, css_ref, m_ref, g1_ref, be1_ref,
               g2_ref, be2_ref, o_ref):
        total = jnp.sum(cs_ref[...])
        mu1 = total / ND
        ssq = jnp.sum(css_ref[...])
        var1 = (ssq - ND * mu1 * mu1) / (ND - 1.0)
        rstd1 = 1.0 / jnp.sqrt(var1 + _EPS)
        a = rstd1 * g1_ref[...]
        c = be1_ref[...] + m_ref[...] - mu1 * a
        sz = jnp.sum(a * cs_ref[...] + N * c)
        szz = jnp.sum(a * a * css_ref[...] + 2.0 * a * c * cs_ref[...]
                      + N * c * c)
        mu2 = sz / ND
        var2 = (szz - ND * mu2 * mu2) / (ND - 1.0)
        rstd2 = 1.0 / jnp.sqrt(var2 + _EPS)
        z = y_ref[...] * a + c
        o_ref[...] = (z - mu2) * rstd2 * g2_ref[...] + be2_ref[...]

    out = pl.pallas_call(
        _final,
        grid=(NT,),
        in_specs=[
            pl.BlockSpec((TILE, D), lambda i: (i, 0)),
            pl.BlockSpec((1, D), lambda i: (0, 0)),
            pl.BlockSpec((1, D), lambda i: (0, 0)),
            pl.BlockSpec((1, D), lambda i: (0, 0)),
            pl.BlockSpec((1, D), lambda i: (0, 0)),
            pl.BlockSpec((1, D), lambda i: (0, 0)),
            pl.BlockSpec((1, D), lambda i: (0, 0)),
            pl.BlockSpec((1, D), lambda i: (0, 0)),
        ],
        out_specs=pl.BlockSpec((TILE, D), lambda i: (i, 0)),
        out_shape=jax.ShapeDtypeStruct((N, D), _F32),
    )(y, cs, css, m, g1r, be1r, g2r, be2r)

    return out.reshape(Bb, N, D)


# speedup vs baseline: 1.0878x; 1.0878x over previous
import math

import jax
import jax.numpy as jnp
from jax import lax
from jax.experimental import pallas as pl
from jax.experimental.pallas import tpu as pltpu

_EPS = 1e-12
_F32 = jnp.float32
_BF16 = jnp.bfloat16


def _dot_t(a, b):
    return lax.dot_general(a, b, (((1,), (1,)), ((), ())),
                           preferred_element_type=_F32)


def _gelu(v):
    inner = math.sqrt(2.0 / math.pi) * (v + 0.044715 * v * v * v)
    return 0.5 * v * (1.0 + jnp.tanh(inner))


def kernel(x, W1w, W1b, W2w, W2b, Wfc, bfc, Wproj, bproj, Wr, br, g1, be1, g2, be2):
    Bb, N, D = x.shape
    E, F, _ = Wfc.shape
    H = 12
    DH = D // H
    K = 2
    TILE = 256
    NT = N // TILE
    ND = float(N * D)
    scale = 1.0 / math.sqrt(float(N))

    x2 = x.reshape(N, D)
    b1r = W1b.reshape(1, 3 * D)
    b2r = W2b.reshape(1, D)
    g1r, be1r = g1.reshape(1, D), be1.reshape(1, D)
    g2r, be2r = g2.reshape(1, D), be2.reshape(1, D)
    brr = br.reshape(1, E)

    def _qkv(x_ref, w1_ref, b1_ref, v_ref, s_ref):
        i = pl.program_id(0)
        xp = _dot_t(x_ref[...], w1_ref[...]) + b1_ref[...]
        v_ref[...] = xp[:, 2 * D:].astype(_BF16)

        @pl.when(i == 0)
        def _():
            s_ref[...] = jnp.zeros_like(s_ref)

        xpb = xp.astype(_BF16)
        for h in range(H):
            qh = xpb[:, h * DH:(h + 1) * DH]
            kh = xpb[:, D + h * DH:D + (h + 1) * DH]
            s_ref[h] += lax.dot_general(qh, kh, (((0,), (0,)), ((), ())),
                                        preferred_element_type=_F32)

    V, S = pl.pallas_call(
        _qkv,
        grid=(NT,),
        in_specs=[
            pl.BlockSpec((TILE, D), lambda i: (i, 0)),
            pl.BlockSpec((3 * D, D), lambda i: (0, 0)),
            pl.BlockSpec((1, 3 * D), lambda i: (0, 0)),
        ],
        out_specs=[
            pl.BlockSpec((TILE, D), lambda i: (i, 0)),
            pl.BlockSpec((H, DH, DH), lambda i: (0, 0, 0)),
        ],
        out_shape=[
            jax.ShapeDtypeStruct((N, D), _BF16),
            jax.ShapeDtypeStruct((H, DH, DH), _F32),
        ],
    )(x2, W1w, b1r)

    def _attn(s_ref, v_ref, o_ref):
        w = jax.nn.softmax(s_ref[...] * scale, axis=-1).astype(_BF16)
        vt = v_ref[...]
        for h in range(H):
            oh = lax.dot_general(w[h], vt[:, h * DH:(h + 1) * DH],
                                 (((1,), (1,)), ((), ())),
                                 preferred_element_type=_F32)
            o_ref[h * DH:(h + 1) * DH, :] = oh.astype(_BF16)

    out_hdn = pl.pallas_call(
        _attn,
        grid=(NT,),
        in_specs=[
            pl.BlockSpec((H, DH, DH), lambda i: (0, 0, 0)),
            pl.BlockSpec((TILE, D), lambda i: (i, 0)),
        ],
        out_specs=pl.BlockSpec((D, TILE), lambda i: (0, i)),
        out_shape=jax.ShapeDtypeStruct((D, N), _BF16),
    )(S, V)

    att = out_hdn.reshape(N, D)

    def _proj(a_ref, w2_ref, b2_ref, x_ref, g1_ref, be1_ref, wr_ref, br_ref,
              y_ref, cs_ref, css_ref, stats_ref, idx_ref):
        i = pl.program_id(0)
        y = _dot_t(a_ref[...], w2_ref[...]) + b2_ref[...] + x_ref[...]
        y_ref[...] = y

        @pl.when(i == 0)
        def _():
            cs_ref[...] = jnp.zeros_like(cs_ref)
            css_ref[...] = jnp.zeros_like(css_ref)

        cs_ref[...] += jnp.sum(y, axis=0, keepdims=True)
        css_ref[...] += jnp.sum(y * y, axis=0, keepdims=True)

        @pl.when(i == NT - 1)
        def _():
            total = jnp.sum(cs_ref[...])
            mu = total / ND
            ssq = jnp.sum(css_ref[...])
            var = (ssq - ND * mu * mu) / (ND - 1.0)
            rstd = 1.0 / jnp.sqrt(var + _EPS)
            colmean_x1 = ((cs_ref[...] / N) - mu) * rstd * g1_ref[...] \
                + be1_ref[...]
            logits = _dot_t(colmean_x1, wr_ref[...]) + br_ref[...]
            probs = jax.nn.softmax(logits, axis=-1)
            iota_e = lax.broadcasted_iota(jnp.int32, (1, E), 1)
            v0 = jnp.max(probs, axis=1, keepdims=True)
            i0 = jnp.argmax(probs, axis=1).astype(jnp.int32)[:, None]
            masked = jnp.where(iota_e == i0, -jnp.inf, probs)
            v1 = jnp.max(masked, axis=1, keepdims=True)
            i1 = jnp.argmax(masked, axis=1).astype(jnp.int32)[:, None]
            iota4 = lax.broadcasted_iota(jnp.int32, (1, 4), 1)
            stats_ref[...] = jnp.where(
                iota4 == 0, mu,
                jnp.where(iota4 == 1, rstd, jnp.where(iota4 == 2, v0, v1)))
            iota2 = lax.broadcasted_iota(jnp.int32, (1, K), 1)
            idx_ref[...] = jnp.where(iota2 == 0, i0, i1)

    y, cs, css, stats, idx2 = pl.pallas_call(
        _proj,
        grid=(NT,),
        in_specs=[
            pl.BlockSpec((TILE, D), lambda i: (i, 0)),
            pl.BlockSpec((D, D), lambda i: (0, 0)),
            pl.BlockSpec((1, D), lambda i: (0, 0)),
            pl.BlockSpec((TILE, D), lambda i: (i, 0)),
            pl.BlockSpec((1, D), lambda i: (0, 0)),
            pl.BlockSpec((1, D), lambda i: (0, 0)),
            pl.BlockSpec((E, D), lambda i: (0, 0)),
            pl.BlockSpec((1, E), lambda i: (0, 0)),
        ],
        out_specs=[
            pl.BlockSpec((TILE, D), lambda i: (i, 0)),
            pl.BlockSpec((1, D), lambda i: (0, 0)),
            pl.BlockSpec((1, D), lambda i: (0, 0)),
            pl.BlockSpec((1, 4), lambda i: (0, 0)),
            pl.BlockSpec((1, K), lambda i: (0, 0)),
        ],
        out_shape=[
            jax.ShapeDtypeStruct((N, D), _F32),
            jax.ShapeDtypeStruct((1, D), _F32),
            jax.ShapeDtypeStruct((1, D), _F32),
            jax.ShapeDtypeStruct((1, 4), _F32),
            jax.ShapeDtypeStruct((1, K), jnp.int32),
        ],
    )(att, W2w, b2r, x2, g1r, be1r, Wr, brr)

    idx_flat = idx2.reshape(K)
    y01 = y[:K].reshape(K, 1, D)
    bfc3 = bfc.reshape(E, 1, F)
    bproj3 = bproj.reshape(E, 1, D)

    FT = 768
    NF = F // FT

    def _expert(idx_ref, y01_ref, stats_ref, g1_ref, be1_ref,
                wfc_ref, bfc_ref, wproj_ref, bproj_ref, m_ref):
        j = pl.program_id(0)
        f = pl.program_id(1)
        mu = stats_ref[:, 0:1]
        rstd = stats_ref[:, 1:2]
        val = jnp.where(j == 0, stats_ref[:, 2:3], stats_ref[:, 3:4])
        x1j = (y01_ref[0] - mu) * rstd * g1_ref[...] + be1_ref[...]
        h = _gelu(_dot_t(x1j, wfc_ref[0]) + bfc_ref[0])
        o = _dot_t(h, wproj_ref[0])

        @pl.when((j == 0) & (f == 0))
        def _():
            m_ref[...] = jnp.zeros_like(m_ref)

        m_ref[...] += val * o

        @pl.when(f == 0)
        def _():
            m_ref[...] += val * bproj_ref[0]

    m = pl.pallas_call(
        _expert,
        grid_spec=pltpu.PrefetchScalarGridSpec(
            num_scalar_prefetch=1,
            grid=(K, NF),
            in_specs=[
                pl.BlockSpec((1, 1, D), lambda j, f, idx: (j, 0, 0)),
                pl.BlockSpec((1, 4), lambda j, f, idx: (0, 0)),
                pl.BlockSpec((1, D), lambda j, f, idx: (0, 0)),
                pl.BlockSpec((1, D), lambda j, f, idx: (0, 0)),
                pl.BlockSpec((1, FT, D), lambda j, f, idx: (idx[j], f, 0)),
                pl.BlockSpec((1, 1, FT), lambda j, f, idx: (idx[j], 0, f)),
                pl.BlockSpec((1, D, FT), lambda j, f, idx: (idx[j], 0, f)),
                pl.BlockSpec((1, 1, D), lambda j, f, idx: (idx[j], 0, 0)),
            ],
            out_specs=pl.BlockSpec((1, D), lambda j, f, idx: (0, 0)),
        ),
        out_shape=jax.ShapeDtypeStruct((1, D), _F32),
    )(idx_flat, y01, stats, g1r, be1r, Wfc, bfc3, Wproj, bproj3)

    def _final(y_ref, cs_ref, css_ref, m_ref, g1_ref, be1_ref,
               g2_ref, be2_ref, o_ref):
        total = jnp.sum(cs_ref[...])
        mu1 = total / ND
        ssq = jnp.sum(css_ref[...])
        var1 = (ssq - ND * mu1 * mu1) / (ND - 1.0)
        rstd1 = 1.0 / jnp.sqrt(var1 + _EPS)
        a = rstd1 * g1_ref[...]
        c = be1_ref[...] + m_ref[...] - mu1 * a
        sz = jnp.sum(a * cs_ref[...] + N * c)
        szz = jnp.sum(a * a * css_ref[...] + 2.0 * a * c * cs_ref[...]
                      + N * c * c)
        mu2 = sz / ND
        var2 = (szz - ND * mu2 * mu2) / (ND - 1.0)
        rstd2 = 1.0 / jnp.sqrt(var2 + _EPS)
        z = y_ref[...] * a + c
        o_ref[...] = (z - mu2) * rstd2 * g2_ref[...] + be2_ref[...]

    out = pl.pallas_call(
        _final,
        grid=(NT,),
        in_specs=[
            pl.BlockSpec((TILE, D), lambda i: (i, 0)),
            pl.BlockSpec((1, D), lambda i: (0, 0)),
            pl.BlockSpec((1, D), lambda i: (0, 0)),
            pl.BlockSpec((1, D), lambda i: (0, 0)),
            pl.BlockSpec((1, D), lambda i: (0, 0)),
            pl.BlockSpec((1, D), lambda i: (0, 0)),
            pl.BlockSpec((1, D), lambda i: (0, 0)),
            pl.BlockSpec((1, D), lambda i: (0, 0)),
        ],
        out_specs=pl.BlockSpec((TILE, D), lambda i: (i, 0)),
        out_shape=jax.ShapeDtypeStruct((N, D), _F32),
    )(y, cs, css, m, g1r, be1r, g2r, be2r)

    return out.reshape(Bb, N, D)


# speedup vs baseline: 1.1482x; 1.0555x over previous
import math

import jax
import jax.numpy as jnp
from jax import lax
from jax.experimental import pallas as pl
from jax.experimental.pallas import tpu as pltpu

_EPS = 1e-12
_F32 = jnp.float32
_BF16 = jnp.bfloat16


def _dot_t(a, b):
    return lax.dot_general(a, b, (((1,), (1,)), ((), ())),
                           preferred_element_type=_F32)


def _gelu(v):
    inner = math.sqrt(2.0 / math.pi) * (v + 0.044715 * v * v * v)
    return 0.5 * v * (1.0 + jnp.tanh(inner))


def kernel(x, W1w, W1b, W2w, W2b, Wfc, bfc, Wproj, bproj, Wr, br, g1, be1, g2, be2):
    Bb, N, D = x.shape
    E, F, _ = Wfc.shape
    H = 12
    DH = D // H
    K = 2
    TILE = 256
    NT = N // TILE
    ND = float(N * D)
    scale = 1.0 / math.sqrt(float(N))

    x2 = x.reshape(N, D)
    b1r = W1b.reshape(1, 3 * D)
    b2r = W2b.reshape(1, D)
    g1r, be1r = g1.reshape(1, D), be1.reshape(1, D)
    g2r, be2r = g2.reshape(1, D), be2.reshape(1, D)
    brr = br.reshape(1, E)

    def _qkv(x_ref, w1_ref, b1_ref, v_ref, s_ref):
        i = pl.program_id(0)
        xp = _dot_t(x_ref[...], w1_ref[...]) + b1_ref[...]
        v_ref[...] = xp[:, 2 * D:]

        @pl.when(i == 0)
        def _():
            s_ref[...] = jnp.zeros_like(s_ref)

        xpb = xp
        for h in range(H):
            qh = xpb[:, h * DH:(h + 1) * DH]
            kh = xpb[:, D + h * DH:D + (h + 1) * DH]
            s_ref[h] += lax.dot_general(qh, kh, (((0,), (0,)), ((), ())),
                                        preferred_element_type=_F32)

    V, S = pl.pallas_call(
        _qkv,
        grid=(NT,),
        in_specs=[
            pl.BlockSpec((TILE, D), lambda i: (i, 0)),
            pl.BlockSpec((3 * D, D), lambda i: (0, 0)),
            pl.BlockSpec((1, 3 * D), lambda i: (0, 0)),
        ],
        out_specs=[
            pl.BlockSpec((TILE, D), lambda i: (i, 0)),
            pl.BlockSpec((H, DH, DH), lambda i: (0, 0, 0)),
        ],
        out_shape=[
            jax.ShapeDtypeStruct((N, D), _F32),
            jax.ShapeDtypeStruct((H, DH, DH), _F32),
        ],
    )(x2, W1w, b1r)

    def _attn(s_ref, v_ref, o_ref):
        w = jax.nn.softmax(s_ref[...] * scale, axis=-1)
        vt = v_ref[...]
        for h in range(H):
            oh = lax.dot_general(w[h], vt[:, h * DH:(h + 1) * DH],
                                 (((1,), (1,)), ((), ())),
                                 preferred_element_type=_F32)
            o_ref[h * DH:(h + 1) * DH, :] = oh

    out_hdn = pl.pallas_call(
        _attn,
        grid=(NT,),
        in_specs=[
            pl.BlockSpec((H, DH, DH), lambda i: (0, 0, 0)),
            pl.BlockSpec((TILE, D), lambda i: (i, 0)),
        ],
        out_specs=pl.BlockSpec((D, TILE), lambda i: (0, i)),
        out_shape=jax.ShapeDtypeStruct((D, N), _F32),
    )(S, V)

    att = out_hdn.reshape(N, D)

    def _proj(a_ref, w2_ref, b2_ref, x_ref, g1_ref, be1_ref, wr_ref, br_ref,
              y_ref, cs_ref, css_ref, stats_ref, idx_ref):
        i = pl.program_id(0)
        y = _dot_t(a_ref[...], w2_ref[...]) + b2_ref[...] + x_ref[...]
        y_ref[...] = y

        @pl.when(i == 0)
        def _():
            cs_ref[...] = jnp.zeros_like(cs_ref)
            css_ref[...] = jnp.zeros_like(css_ref)

        cs_ref[...] += jnp.sum(y, axis=0, keepdims=True)
        css_ref[...] += jnp.sum(y * y, axis=0, keepdims=True)

        @pl.when(i == NT - 1)
        def _():
            total = jnp.sum(cs_ref[...])
            mu = total / ND
            ssq = jnp.sum(css_ref[...])
            var = (ssq - ND * mu * mu) / (ND - 1.0)
            rstd = 1.0 / jnp.sqrt(var + _EPS)
            colmean_x1 = ((cs_ref[...] / N) - mu) * rstd * g1_ref[...] \
                + be1_ref[...]
            logits = _dot_t(colmean_x1, wr_ref[...]) + br_ref[...]
            probs = jax.nn.softmax(logits, axis=-1)
            iota_e = lax.broadcasted_iota(jnp.int32, (1, E), 1)
            v0 = jnp.max(probs, axis=1, keepdims=True)
            i0 = jnp.argmax(probs, axis=1).astype(jnp.int32)[:, None]
            masked = jnp.where(iota_e == i0, -jnp.inf, probs)
            v1 = jnp.max(masked, axis=1, keepdims=True)
            i1 = jnp.argmax(masked, axis=1).astype(jnp.int32)[:, None]
            iota4 = lax.broadcasted_iota(jnp.int32, (1, 4), 1)
            stats_ref[...] = jnp.where(
                iota4 == 0, mu,
                jnp.where(iota4 == 1, rstd, jnp.where(iota4 == 2, v0, v1)))
            iota2 = lax.broadcasted_iota(jnp.int32, (1, K), 1)
            idx_ref[...] = jnp.where(iota2 == 0, i0, i1)

    y, cs, css, stats, idx2 = pl.pallas_call(
        _proj,
        grid=(NT,),
        in_specs=[
            pl.BlockSpec((TILE, D), lambda i: (i, 0)),
            pl.BlockSpec((D, D), lambda i: (0, 0)),
            pl.BlockSpec((1, D), lambda i: (0, 0)),
            pl.BlockSpec((TILE, D), lambda i: (i, 0)),
            pl.BlockSpec((1, D), lambda i: (0, 0)),
            pl.BlockSpec((1, D), lambda i: (0, 0)),
            pl.BlockSpec((E, D), lambda i: (0, 0)),
            pl.BlockSpec((1, E), lambda i: (0, 0)),
        ],
        out_specs=[
            pl.BlockSpec((TILE, D), lambda i: (i, 0)),
            pl.BlockSpec((1, D), lambda i: (0, 0)),
            pl.BlockSpec((1, D), lambda i: (0, 0)),
            pl.BlockSpec((1, 4), lambda i: (0, 0)),
            pl.BlockSpec((1, K), lambda i: (0, 0)),
        ],
        out_shape=[
            jax.ShapeDtypeStruct((N, D), _F32),
            jax.ShapeDtypeStruct((1, D), _F32),
            jax.ShapeDtypeStruct((1, D), _F32),
            jax.ShapeDtypeStruct((1, 4), _F32),
            jax.ShapeDtypeStruct((1, K), jnp.int32),
        ],
    )(att, W2w, b2r, x2, g1r, be1r, Wr, brr)

    idx_flat = idx2.reshape(K)
    y01 = y[:K].reshape(K, 1, D)
    bfc3 = bfc.reshape(E, 1, F)
    bproj3 = bproj.reshape(E, 1, D)

    FT = 768
    NF = F // FT

    def _expert(idx_ref, y01_ref, stats_ref, g1_ref, be1_ref,
                wfc_ref, bfc_ref, wproj_ref, bproj_ref, m_ref):
        j = pl.program_id(0)
        f = pl.program_id(1)
        mu = stats_ref[:, 0:1]
        rstd = stats_ref[:, 1:2]
        val = jnp.where(j == 0, stats_ref[:, 2:3], stats_ref[:, 3:4])
        x1j = (y01_ref[0] - mu) * rstd * g1_ref[...] + be1_ref[...]
        h = _gelu(_dot_t(x1j, wfc_ref[0]) + bfc_ref[0])
        o = _dot_t(h, wproj_ref[0])

        @pl.when((j == 0) & (f == 0))
        def _():
            m_ref[...] = jnp.zeros_like(m_ref)

        m_ref[...] += val * o

        @pl.when(f == 0)
        def _():
            m_ref[...] += val * bproj_ref[0]

    m = pl.pallas_call(
        _expert,
        grid_spec=pltpu.PrefetchScalarGridSpec(
            num_scalar_prefetch=1,
            grid=(K, NF),
            in_specs=[
                pl.BlockSpec((1, 1, D), lambda j, f, idx: (j, 0, 0)),
                pl.BlockSpec((1, 4), lambda j, f, idx: (0, 0)),
                pl.BlockSpec((1, D), lambda j, f, idx: (0, 0)),
                pl.BlockSpec((1, D), lambda j, f, idx: (0, 0)),
                pl.BlockSpec((1, FT, D), lambda j, f, idx: (idx[j], f, 0)),
                pl.BlockSpec((1, 1, FT), lambda j, f, idx: (idx[j], 0, f)),
                pl.BlockSpec((1, D, FT), lambda j, f, idx: (idx[j], 0, f)),
                pl.BlockSpec((1, 1, D), lambda j, f, idx: (idx[j], 0, 0)),
            ],
            out_specs=pl.BlockSpec((1, D), lambda j, f, idx: (0, 0)),
        ),
        out_shape=jax.ShapeDtypeStruct((1, D), _F32),
    )(idx_flat, y01, stats, g1r, be1r, Wfc, bfc3, Wproj, bproj3)

    def _final(y_ref, cs_ref, css_ref, m_ref, g1_ref, be1_ref,
               g2_ref, be2_ref, o_ref):
        total = jnp.sum(cs_ref[...])
        mu1 = total / ND
        ssq = jnp.sum(css_ref[...])
        var1 = (ssq - ND * mu1 * mu1) / (ND - 1.0)
        rstd1 = 1.0 / jnp.sqrt(var1 + _EPS)
        a = rstd1 * g1_ref[...]
        c = be1_ref[...] + m_ref[...] - mu1 * a
        sz = jnp.sum(a * cs_ref[...] + N * c)
        szz = jnp.sum(a * a * css_ref[...] + 2.0 * a * c * cs_ref[...]
                      + N * c * c)
        mu2 = sz / ND
        var2 = (szz - ND * mu2 * mu2) / (ND - 1.0)
        rstd2 = 1.0 / jnp.sqrt(var2 + _EPS)
        z = y_ref[...] * a + c
        o_ref[...] = (z - mu2) * rstd2 * g2_ref[...] + be2_ref[...]

    out = pl.pallas_call(
        _final,
        grid=(NT,),
        in_specs=[
            pl.BlockSpec((TILE, D), lambda i: (i, 0)),
            pl.BlockSpec((1, D), lambda i: (0, 0)),
            pl.BlockSpec((1, D), lambda i: (0, 0)),
            pl.BlockSpec((1, D), lambda i: (0, 0)),
            pl.BlockSpec((1, D), lambda i: (0, 0)),
            pl.BlockSpec((1, D), lambda i: (0, 0)),
            pl.BlockSpec((1, D), lambda i: (0, 0)),
            pl.BlockSpec((1, D), lambda i: (0, 0)),
        ],
        out_specs=pl.BlockSpec((TILE, D), lambda i: (i, 0)),
        out_shape=jax.ShapeDtypeStruct((N, D), _F32),
    )(y, cs, css, m, g1r, be1r, g2r, be2r)

    return out.reshape(Bb, N, D)


# speedup vs baseline: 1.3205x; 1.1501x over previous
import math

import jax
import jax.numpy as jnp
from jax import lax
from jax.experimental import pallas as pl
from jax.experimental.pallas import tpu as pltpu

_EPS = 1e-12
_F32 = jnp.float32
_BF16 = jnp.bfloat16


def _dot_t(a, b):
    return lax.dot_general(a, b, (((1,), (1,)), ((), ())),
                           preferred_element_type=_F32)


def _gelu(v):
    inner = math.sqrt(2.0 / math.pi) * (v + 0.044715 * v * v * v)
    return 0.5 * v * (1.0 + jnp.tanh(inner))


def kernel(x, W1w, W1b, W2w, W2b, Wfc, bfc, Wproj, bproj, Wr, br, g1, be1, g2, be2):
    Bb, N, D = x.shape
    E, F, _ = Wfc.shape
    H = 12
    DH = D // H
    K = 2
    TILE = 256
    NT = N // TILE
    ND = float(N * D)
    scale = 1.0 / math.sqrt(float(N))

    x2 = x.reshape(N, D)
    b1r = W1b.reshape(1, 3 * D)
    b2r = W2b.reshape(1, D)
    g1r, be1r = g1.reshape(1, D), be1.reshape(1, D)
    g2r, be2r = g2.reshape(1, D), be2.reshape(1, D)
    brr = br.reshape(1, E)

    def _qkv(x_ref, w1_ref, b1_ref, v_ref, s_ref):
        i = pl.program_id(0)
        xp = _dot_t(x_ref[...], w1_ref[...]) + b1_ref[...]
        for h in range(H):
            v_ref[h] = xp[:, 2 * D + h * DH:2 * D + (h + 1) * DH]

        @pl.when(i == 0)
        def _():
            s_ref[...] = jnp.zeros_like(s_ref)

        xpb = xp
        for h in range(H):
            qh = xpb[:, h * DH:(h + 1) * DH]
            kh = xpb[:, D + h * DH:D + (h + 1) * DH]
            s_ref[h] += lax.dot_general(qh, kh, (((0,), (0,)), ((), ())),
                                        preferred_element_type=_F32)

    V, S = pl.pallas_call(
        _qkv,
        grid=(NT,),
        in_specs=[
            pl.BlockSpec((TILE, D), lambda i: (i, 0)),
            pl.BlockSpec((3 * D, D), lambda i: (0, 0)),
            pl.BlockSpec((1, 3 * D), lambda i: (0, 0)),
        ],
        out_specs=[
            pl.BlockSpec((H, TILE, DH), lambda i: (0, i, 0)),
            pl.BlockSpec((H, DH, DH), lambda i: (0, 0, 0)),
        ],
        out_shape=[
            jax.ShapeDtypeStruct((H, N, DH), _F32),
            jax.ShapeDtypeStruct((H, DH, DH), _F32),
        ],
    )(x2, W1w, b1r)

    RS = (H * DH * DH) // (2 * TILE)

    def _proj(sa_ref, sb_ref, va_ref, vb_ref, w2_ref, b2_ref, x_ref,
              g1_ref, be1_ref, wr_ref, br_ref,
              y_ref, cs_ref, css_ref, stats_ref, idx_ref):
        i = pl.program_id(0)
        wA = jax.nn.softmax(sa_ref[0] * scale, axis=-1)
        wB = jax.nn.softmax(sb_ref[0] * scale, axis=-1)
        vA = va_ref[0]
        vB = vb_ref[0]

        def combine(wa, wb):
            ra = lax.dot_general(wa, vA, (((1,), (1,)), ((), ())),
                                 preferred_element_type=_F32)
            rb = lax.dot_general(wb, vB, (((1,), (1,)), ((), ())),
                                 preferred_element_type=_F32)
            r = jnp.concatenate([ra, rb], axis=0)
            return jnp.reshape(r, (TILE, D))

        att_even = combine(wA, wB[:DH // 2])
        att_odd = combine(wA[DH // 2:], wB)
        att = jnp.where(i % 2 == 0, att_even, att_odd)
        y = _dot_t(att, w2_ref[...]) + b2_ref[...] + x_ref[...]
        y_ref[...] = y

        @pl.when(i == 0)
        def _():
            cs_ref[...] = jnp.zeros_like(cs_ref)
            css_ref[...] = jnp.zeros_like(css_ref)

        cs_ref[...] += jnp.sum(y, axis=0, keepdims=True)
        css_ref[...] += jnp.sum(y * y, axis=0, keepdims=True)

        @pl.when(i == NT - 1)
        def _():
            total = jnp.sum(cs_ref[...])
            mu = total / ND
            ssq = jnp.sum(css_ref[...])
            var = (ssq - ND * mu * mu) / (ND - 1.0)
            rstd = 1.0 / jnp.sqrt(var + _EPS)
            colmean_x1 = ((cs_ref[...] / N) - mu) * rstd * g1_ref[...] \
                + be1_ref[...]
            logits = _dot_t(colmean_x1, wr_ref[...]) + br_ref[...]
            probs = jax.nn.softmax(logits, axis=-1)
            iota_e = lax.broadcasted_iota(jnp.int32, (1, E), 1)
            v0 = jnp.max(probs, axis=1, keepdims=True)
            i0 = jnp.argmax(probs, axis=1).astype(jnp.int32)[:, None]
            masked = jnp.where(iota_e == i0, -jnp.inf, probs)
            v1 = jnp.max(masked, axis=1, keepdims=True)
            i1 = jnp.argmax(masked, axis=1).astype(jnp.int32)[:, None]
            iota4 = lax.broadcasted_iota(jnp.int32, (1, 4), 1)
            stats_ref[...] = jnp.where(
                iota4 == 0, mu,
                jnp.where(iota4 == 1, rstd, jnp.where(iota4 == 2, v0, v1)))
            iota2 = lax.broadcasted_iota(jnp.int32, (1, K), 1)
            idx_ref[...] = jnp.where(iota2 == 0, i0, i1)

    y, cs, css, stats, idx2 = pl.pallas_call(
        _proj,
        grid=(NT,),
        in_specs=[
            pl.BlockSpec((1, DH, DH), lambda i: ((3 * i) // 2, 0, 0)),
            pl.BlockSpec((1, DH, DH), lambda i: ((3 * i) // 2 + 1, 0, 0)),
            pl.BlockSpec((1, N, DH), lambda i: ((3 * i) // 2, 0, 0)),
            pl.BlockSpec((1, N, DH), lambda i: ((3 * i) // 2 + 1, 0, 0)),
            pl.BlockSpec((D, D), lambda i: (0, 0)),
            pl.BlockSpec((1, D), lambda i: (0, 0)),
            pl.BlockSpec((TILE, D), lambda i: (i, 0)),
            pl.BlockSpec((1, D), lambda i: (0, 0)),
            pl.BlockSpec((1, D), lambda i: (0, 0)),
            pl.BlockSpec((E, D), lambda i: (0, 0)),
            pl.BlockSpec((1, E), lambda i: (0, 0)),
        ],
        out_specs=[
            pl.BlockSpec((TILE, D), lambda i: (i, 0)),
            pl.BlockSpec((1, D), lambda i: (0, 0)),
            pl.BlockSpec((1, D), lambda i: (0, 0)),
            pl.BlockSpec((1, 4), lambda i: (0, 0)),
            pl.BlockSpec((1, K), lambda i: (0, 0)),
        ],
        out_shape=[
            jax.ShapeDtypeStruct((N, D), _F32),
            jax.ShapeDtypeStruct((1, D), _F32),
            jax.ShapeDtypeStruct((1, D), _F32),
            jax.ShapeDtypeStruct((1, 4), _F32),
            jax.ShapeDtypeStruct((1, K), jnp.int32),
        ],
    )(S, S, V, V, W2w, b2r, x2, g1r, be1r, Wr, brr)

    idx_flat = idx2.reshape(K)
    y01 = y[:K].reshape(K, 1, D)
    bfc3 = bfc.reshape(E, 1, F)
    bproj3 = bproj.reshape(E, 1, D)

    FT = 768
    NF = F // FT

    def _expert(idx_ref, y01_ref, stats_ref, g1_ref, be1_ref,
                wfc_ref, bfc_ref, wproj_ref, bproj_ref, m_ref):
        j = pl.program_id(0)
        f = pl.program_id(1)
        mu = stats_ref[:, 0:1]
        rstd = stats_ref[:, 1:2]
        val = jnp.where(j == 0, stats_ref[:, 2:3], stats_ref[:, 3:4])
        x1j = (y01_ref[0] - mu) * rstd * g1_ref[...] + be1_ref[...]
        h = _gelu(_dot_t(x1j, wfc_ref[0]) + bfc_ref[0])
        o = _dot_t(h, wproj_ref[0])

        @pl.when((j == 0) & (f == 0))
        def _():
            m_ref[...] = jnp.zeros_like(m_ref)

        m_ref[...] += val * o

        @pl.when(f == 0)
        def _():
            m_ref[...] += val * bproj_ref[0]

    m = pl.pallas_call(
        _expert,
        grid_spec=pltpu.PrefetchScalarGridSpec(
            num_scalar_prefetch=1,
            grid=(K, NF),
            in_specs=[
                pl.BlockSpec((1, 1, D), lambda j, f, idx: (j, 0, 0)),
                pl.BlockSpec((1, 4), lambda j, f, idx: (0, 0)),
                pl.BlockSpec((1, D), lambda j, f, idx: (0, 0)),
                pl.BlockSpec((1, D), lambda j, f, idx: (0, 0)),
                pl.BlockSpec((1, FT, D), lambda j, f, idx: (idx[j], f, 0)),
                pl.BlockSpec((1, 1, FT), lambda j, f, idx: (idx[j], 0, f)),
                pl.BlockSpec((1, D, FT), lambda j, f, idx: (idx[j], 0, f)),
                pl.BlockSpec((1, 1, D), lambda j, f, idx: (idx[j], 0, 0)),
            ],
            out_specs=pl.BlockSpec((1, D), lambda j, f, idx: (0, 0)),
        ),
        out_shape=jax.ShapeDtypeStruct((1, D), _F32),
    )(idx_flat, y01, stats, g1r, be1r, Wfc, bfc3, Wproj, bproj3)

    def _final(y_ref, cs_ref, css_ref, m_ref, g1_ref, be1_ref,
               g2_ref, be2_ref, o_ref):
        total = jnp.sum(cs_ref[...])
        mu1 = total / ND
        ssq = jnp.sum(css_ref[...])
        var1 = (ssq - ND * mu1 * mu1) / (ND - 1.0)
        rstd1 = 1.0 / jnp.sqrt(var1 + _EPS)
        a = rstd1 * g1_ref[...]
        c = be1_ref[...] + m_ref[...] - mu1 * a
        sz = jnp.sum(a * cs_ref[...] + N * c)
        szz = jnp.sum(a * a * css_ref[...] + 2.0 * a * c * cs_ref[...]
                      + N * c * c)
        mu2 = sz / ND
        var2 = (szz - ND * mu2 * mu2) / (ND - 1.0)
        rstd2 = 1.0 / jnp.sqrt(var2 + _EPS)
        z = y_ref[...] * a + c
        o_ref[...] = (z - mu2) * rstd2 * g2_ref[...] + be2_ref[...]

    out = pl.pallas_call(
        _final,
        grid=(NT,),
        in_specs=[
            pl.BlockSpec((TILE, D), lambda i: (i, 0)),
            pl.BlockSpec((1, D), lambda i: (0, 0)),
            pl.BlockSpec((1, D), lambda i: (0, 0)),
            pl.BlockSpec((1, D), lambda i: (0, 0)),
            pl.BlockSpec((1, D), lambda i: (0, 0)),
            pl.BlockSpec((1, D), lambda i: (0, 0)),
            pl.BlockSpec((1, D), lambda i: (0, 0)),
            pl.BlockSpec((1, D), lambda i: (0, 0)),
        ],
        out_specs=pl.BlockSpec((TILE, D), lambda i: (i, 0)),
        out_shape=jax.ShapeDtypeStruct((N, D), _F32),
    )(y, cs, css, m, g1r, be1r, g2r, be2r)

    return out.reshape(Bb, N, D)


# speedup vs baseline: 1.5337x; 1.1614x over previous
import math

import jax
import jax.numpy as jnp
from jax import lax
from jax.experimental import pallas as pl
from jax.experimental.pallas import tpu as pltpu

_EPS = 1e-12
_F32 = jnp.float32
_BF16 = jnp.bfloat16


def _dot_t(a, b):
    return lax.dot_general(a, b, (((1,), (1,)), ((), ())),
                           preferred_element_type=_F32)


def _gelu(v):
    inner = math.sqrt(2.0 / math.pi) * (v + 0.044715 * v * v * v)
    return 0.5 * v * (1.0 + jnp.tanh(inner))


def kernel(x, W1w, W1b, W2w, W2b, Wfc, bfc, Wproj, bproj, Wr, br, g1, be1, g2, be2):
    Bb, N, D = x.shape
    E, F, _ = Wfc.shape
    H = 12
    DH = D // H
    K = 2
    TILE = 512
    NT = N // TILE
    ND = float(N * D)
    scale = 1.0 / math.sqrt(float(N))

    x2 = x.reshape(N, D)
    b1r = W1b.reshape(1, 3 * D)
    b2r = W2b.reshape(1, D)
    g1r, be1r = g1.reshape(1, D), be1.reshape(1, D)
    g2r, be2r = g2.reshape(1, D), be2.reshape(1, D)
    brr = br.reshape(1, E)

    def _qkv(x_ref, w1_ref, b1_ref, v_ref, s_ref):
        i = pl.program_id(0)
        xp = _dot_t(x_ref[...], w1_ref[...]) + b1_ref[...]
        for h in range(H):
            v_ref[h] = xp[:, 2 * D + h * DH:2 * D + (h + 1) * DH]

        @pl.when(i == 0)
        def _():
            s_ref[...] = jnp.zeros_like(s_ref)

        xpb = xp
        for h in range(H):
            qh = xpb[:, h * DH:(h + 1) * DH]
            kh = xpb[:, D + h * DH:D + (h + 1) * DH]
            s_ref[h] += lax.dot_general(qh, kh, (((0,), (0,)), ((), ())),
                                        preferred_element_type=_F32)

    V, S = pl.pallas_call(
        _qkv,
        grid=(NT,),
        in_specs=[
            pl.BlockSpec((TILE, D), lambda i: (i, 0)),
            pl.BlockSpec((3 * D, D), lambda i: (0, 0)),
            pl.BlockSpec((1, 3 * D), lambda i: (0, 0)),
        ],
        out_specs=[
            pl.BlockSpec((H, TILE, DH), lambda i: (0, i, 0)),
            pl.BlockSpec((H, DH, DH), lambda i: (0, 0, 0)),
        ],
        out_shape=[
            jax.ShapeDtypeStruct((H, N, DH), _F32),
            jax.ShapeDtypeStruct((H, DH, DH), _F32),
        ],
    )(x2, W1w, b1r)

    def _proj(sa_ref, sb_ref, sc_ref, va_ref, vb_ref, vc_ref,
              w2_ref, b2_ref, x_ref, g1_ref, be1_ref, wr_ref, br_ref,
              y_ref, cs_ref, css_ref, stats_ref, idx_ref):
        i = pl.program_id(0)
        parts = []
        for s_ref, v_ref in ((sa_ref, va_ref), (sb_ref, vb_ref),
                             (sc_ref, vc_ref)):
            w = jax.nn.softmax(s_ref[0] * scale, axis=-1)
            parts.append(lax.dot_general(w, v_ref[0], (((1,), (1,)), ((), ())),
                                         preferred_element_type=_F32))
        r = jnp.concatenate(parts, axis=0)
        att = jnp.reshape(r, (TILE, D))
        y = _dot_t(att, w2_ref[...]) + b2_ref[...] + x_ref[...]
        y_ref[...] = y

        @pl.when(i == 0)
        def _():
            cs_ref[...] = jnp.zeros_like(cs_ref)
            css_ref[...] = jnp.zeros_like(css_ref)

        cs_ref[...] += jnp.sum(y, axis=0, keepdims=True)
        css_ref[...] += jnp.sum(y * y, axis=0, keepdims=True)

        @pl.when(i == NT - 1)
        def _():
            total = jnp.sum(cs_ref[...])
            mu = total / ND
            ssq = jnp.sum(css_ref[...])
            var = (ssq - ND * mu * mu) / (ND - 1.0)
            rstd = 1.0 / jnp.sqrt(var + _EPS)
            colmean_x1 = ((cs_ref[...] / N) - mu) * rstd * g1_ref[...] \
                + be1_ref[...]
            logits = _dot_t(colmean_x1, wr_ref[...]) + br_ref[...]
            probs = jax.nn.softmax(logits, axis=-1)
            iota_e = lax.broadcasted_iota(jnp.int32, (1, E), 1)
            v0 = jnp.max(probs, axis=1, keepdims=True)
            i0 = jnp.argmax(probs, axis=1).astype(jnp.int32)[:, None]
            masked = jnp.where(iota_e == i0, -jnp.inf, probs)
            v1 = jnp.max(masked, axis=1, keepdims=True)
            i1 = jnp.argmax(masked, axis=1).astype(jnp.int32)[:, None]
            iota4 = lax.broadcasted_iota(jnp.int32, (1, 4), 1)
            stats_ref[...] = jnp.where(
                iota4 == 0, mu,
                jnp.where(iota4 == 1, rstd, jnp.where(iota4 == 2, v0, v1)))
            iota2 = lax.broadcasted_iota(jnp.int32, (1, K), 1)
            idx_ref[...] = jnp.where(iota2 == 0, i0, i1)

    y, cs, css, stats, idx2 = pl.pallas_call(
        _proj,
        grid=(NT,),
        in_specs=[
            pl.BlockSpec((1, DH, DH), lambda i: (3 * i, 0, 0)),
            pl.BlockSpec((1, DH, DH), lambda i: (3 * i + 1, 0, 0)),
            pl.BlockSpec((1, DH, DH), lambda i: (3 * i + 2, 0, 0)),
            pl.BlockSpec((1, N, DH), lambda i: (3 * i, 0, 0)),
            pl.BlockSpec((1, N, DH), lambda i: (3 * i + 1, 0, 0)),
            pl.BlockSpec((1, N, DH), lambda i: (3 * i + 2, 0, 0)),
            pl.BlockSpec((D, D), lambda i: (0, 0)),
            pl.BlockSpec((1, D), lambda i: (0, 0)),
            pl.BlockSpec((TILE, D), lambda i: (i, 0)),
            pl.BlockSpec((1, D), lambda i: (0, 0)),
            pl.BlockSpec((1, D), lambda i: (0, 0)),
            pl.BlockSpec((E, D), lambda i: (0, 0)),
            pl.BlockSpec((1, E), lambda i: (0, 0)),
        ],
        out_specs=[
            pl.BlockSpec((TILE, D), lambda i: (i, 0)),
            pl.BlockSpec((1, D), lambda i: (0, 0)),
            pl.BlockSpec((1, D), lambda i: (0, 0)),
            pl.BlockSpec((1, 4), lambda i: (0, 0)),
            pl.BlockSpec((1, K), lambda i: (0, 0)),
        ],
        out_shape=[
            jax.ShapeDtypeStruct((N, D), _F32),
            jax.ShapeDtypeStruct((1, D), _F32),
            jax.ShapeDtypeStruct((1, D), _F32),
            jax.ShapeDtypeStruct((1, 4), _F32),
            jax.ShapeDtypeStruct((1, K), jnp.int32),
        ],
    )(S, S, S, V, V, V, W2w, b2r, x2, g1r, be1r, Wr, brr)

    idx_flat = idx2.reshape(K)
    y01 = y[:K].reshape(K, 1, D)
    bfc3 = bfc.reshape(E, 1, F)
    bproj3 = bproj.reshape(E, 1, D)

    FT = 768
    NF = F // FT

    def _expert(idx_ref, y01_ref, stats_ref, g1_ref, be1_ref,
                wfc_ref, bfc_ref, wproj_ref, bproj_ref, m_ref):
        j = pl.program_id(0)
        f = pl.program_id(1)
        mu = stats_ref[:, 0:1]
        rstd = stats_ref[:, 1:2]
        val = jnp.where(j == 0, stats_ref[:, 2:3], stats_ref[:, 3:4])
        x1j = (y01_ref[0] - mu) * rstd * g1_ref[...] + be1_ref[...]
        h = _gelu(_dot_t(x1j, wfc_ref[0]) + bfc_ref[0])
        o = _dot_t(h, wproj_ref[0])

        @pl.when((j == 0) & (f == 0))
        def _():
            m_ref[...] = jnp.zeros_like(m_ref)

        m_ref[...] += val * o

        @pl.when(f == 0)
        def _():
            m_ref[...] += val * bproj_ref[0]

    m = pl.pallas_call(
        _expert,
        grid_spec=pltpu.PrefetchScalarGridSpec(
            num_scalar_prefetch=1,
            grid=(K, NF),
            in_specs=[
                pl.BlockSpec((1, 1, D), lambda j, f, idx: (j, 0, 0)),
                pl.BlockSpec((1, 4), lambda j, f, idx: (0, 0)),
                pl.BlockSpec((1, D), lambda j, f, idx: (0, 0)),
                pl.BlockSpec((1, D), lambda j, f, idx: (0, 0)),
                pl.BlockSpec((1, FT, D), lambda j, f, idx: (idx[j], f, 0)),
                pl.BlockSpec((1, 1, FT), lambda j, f, idx: (idx[j], 0, f)),
                pl.BlockSpec((1, D, FT), lambda j, f, idx: (idx[j], 0, f)),
                pl.BlockSpec((1, 1, D), lambda j, f, idx: (idx[j], 0, 0)),
            ],
            out_specs=pl.BlockSpec((1, D), lambda j, f, idx: (0, 0)),
        ),
        out_shape=jax.ShapeDtypeStruct((1, D), _F32),
    )(idx_flat, y01, stats, g1r, be1r, Wfc, bfc3, Wproj, bproj3)

    def _final(y_ref, cs_ref, css_ref, m_ref, g1_ref, be1_ref,
               g2_ref, be2_ref, o_ref):
        total = jnp.sum(cs_ref[...])
        mu1 = total / ND
        ssq = jnp.sum(css_ref[...])
        var1 = (ssq - ND * mu1 * mu1) / (ND - 1.0)
        rstd1 = 1.0 / jnp.sqrt(var1 + _EPS)
        a = rstd1 * g1_ref[...]
        c = be1_ref[...] + m_ref[...] - mu1 * a
        sz = jnp.sum(a * cs_ref[...] + N * c)
        szz = jnp.sum(a * a * css_ref[...] + 2.0 * a * c * cs_ref[...]
                      + N * c * c)
        mu2 = sz / ND
        var2 = (szz - ND * mu2 * mu2) / (ND - 1.0)
        rstd2 = 1.0 / jnp.sqrt(var2 + _EPS)
        z = y_ref[...] * a + c
        o_ref[...] = (z - mu2) * rstd2 * g2_ref[...] + be2_ref[...]

    out = pl.pallas_call(
        _final,
        grid=(NT,),
        in_specs=[
            pl.BlockSpec((TILE, D), lambda i: (i, 0)),
            pl.BlockSpec((1, D), lambda i: (0, 0)),
            pl.BlockSpec((1, D), lambda i: (0, 0)),
            pl.BlockSpec((1, D), lambda i: (0, 0)),
            pl.BlockSpec((1, D), lambda i: (0, 0)),
            pl.BlockSpec((1, D), lambda i: (0, 0)),
            pl.BlockSpec((1, D), lambda i: (0, 0)),
            pl.BlockSpec((1, D), lambda i: (0, 0)),
        ],
        out_specs=pl.BlockSpec((TILE, D), lambda i: (i, 0)),
        out_shape=jax.ShapeDtypeStruct((N, D), _F32),
    )(y, cs, css, m, g1r, be1r, g2r, be2r)

    return out.reshape(Bb, N, D)
